# baseline (speedup 1.0000x reference)
import jax
import jax.numpy as jnp
from jax.experimental import pallas as pl
from jax.experimental.pallas import tpu as pltpu

_N = 100000
_K = 64
_BLK = 2000
_NB = _N // _BLK


def _body(w0_ref, xu, xt, xb, wu, wt, wb, uV, bV,
          out_ref, acc_u, acc_t, acc_s, acc_sq, acc_b):
    i = pl.program_id(0)

    @pl.when(i == 0)
    def _init():
        acc_u[...] = jnp.zeros_like(acc_u)
        acc_t[...] = jnp.zeros_like(acc_t)
        acc_s[...] = jnp.zeros_like(acc_s)
        acc_sq[...] = jnp.zeros_like(acc_sq)
        acc_b[...] = jnp.zeros_like(acc_b)

    xu_v = xu[0]
    xt_v = xt[0]
    xb_v = xb[0]
    u_blk = uV[...]
    b_blk = bV[...]
    maskw = (xb_v == 1.0).astype(jnp.float32)

    acc_u[...] += jnp.dot(xu_v, u_blk, preferred_element_type=jnp.float32)
    acc_t[...] += jnp.dot(xt_v, b_blk, preferred_element_type=jnp.float32)
    acc_s[...] += jnp.dot(maskw, b_blk, preferred_element_type=jnp.float32)
    acc_sq[...] += jnp.dot(maskw, b_blk * b_blk,
                           preferred_element_type=jnp.float32)
    wsum = jnp.sum(xu_v * wu[0] + xt_v * wt[0] + xb_v * wb[0])
    acc_b[...] += jnp.reshape(wsum, (1, 1))

    @pl.when(i == _NB - 1)
    def _fin():
        u = acc_u[...]
        t = acc_t[...]
        s = acc_s[...]
        u_t = jnp.sum(u * t)
        t_b = jnp.sum(t * s)
        u_b = jnp.sum(u * s)
        bs = 0.5 * (jnp.sum(s * s) - jnp.sum(acc_sq[...]))
        y = w0_ref[0, 0] + acc_b[0, 0] + u_t + t_b + bs + u_b
        out_ref[...] = jnp.reshape(y, (1, 1))


def _xspec(off):
    return pl.BlockSpec((1, 1, _BLK), lambda i, off=off: (i + off, 0, 0))


_VSPEC = pl.BlockSpec((_BLK, _K), lambda i: (i, 0))


@jax.jit
def _fm(x, w_0, w_bias, u_V, b_V):
    x3 = x.reshape(3 * _NB, 1, _BLK)
    w3 = w_bias.reshape(3 * _NB, 1, _BLK)
    w0 = w_0.reshape(1, 1)
    return pl.pallas_call(
        _body,
        grid=(_NB,),
        in_specs=[
            pl.BlockSpec((1, 1), lambda i: (0, 0)),
            _xspec(0), _xspec(_NB), _xspec(2 * _NB),
            _xspec(0), _xspec(_NB), _xspec(2 * _NB),
            _VSPEC, _VSPEC,
        ],
        out_specs=pl.BlockSpec((1, 1), lambda i: (0, 0)),
        out_shape=jax.ShapeDtypeStruct((1, 1), jnp.float32),
        scratch_shapes=[pltpu.VMEM((1, _K), jnp.float32)] * 4
        + [pltpu.VMEM((1, 1), jnp.float32)],
    )(w0, x3, x3, x3, w3, w3, w3, u_V, b_V)


def kernel(x, delta, pmi, w_0, w_bias, u_V, b_V):
    return _fm(x, w_0, w_bias, u_V, b_V)
